# baseline (speedup 1.0000x reference)
import jax
import jax.numpy as jnp
from jax.experimental import pallas as pl
from jax.experimental.pallas import tpu as pltpu

_B, _C, _K, _H, _W = 64, 512, 64, 64, 64
_HW = _H * _W
_R = _HW // _C
_EPS = 1e-12


def _netvlad_kernel(x2_ref, xf_ref, w_ref, b_ref, cent_ref, out_ref):
    x2 = x2_ref[0]
    u = jax.lax.dot_general(
        w_ref[...], x2, (((1,), (0,)), ((), ())),
        preferred_element_type=jnp.float32)
    ssq = jnp.sum(x2 * x2, axis=0, keepdims=True)
    rnorm = 1.0 / jnp.maximum(jnp.sqrt(ssq), _EPS)
    logits = u * rnorm + b_ref[...]
    m = jnp.max(logits, axis=0, keepdims=True)
    e = jnp.exp(logits - m)
    a = e / jnp.sum(e, axis=0, keepdims=True)

    xf = xf_ref[0]
    sq3 = (xf * xf).reshape(_C, _R, _C)
    ssq8 = jnp.sum(sq3, axis=0)
    rnorm8 = 1.0 / jnp.maximum(jnp.sqrt(ssq8), _EPS)
    xfn = (xf.reshape(_C, _R, _C) * rnorm8[None]).reshape(_HW, _C)

    vlad = jax.lax.dot_general(
        a, xfn, (((1,), (0,)), ((), ())),
        preferred_element_type=jnp.float32)
    vlad = vlad - jnp.sum(a, axis=1, keepdims=True) * cent_ref[...]
    n1 = jnp.sqrt(jnp.sum(vlad * vlad, axis=1, keepdims=True))
    vlad = vlad / jnp.maximum(n1, _EPS)
    n2 = jnp.sqrt(jnp.sum(vlad * vlad))
    out_ref[0] = vlad / jnp.maximum(n2, _EPS)


def kernel(x, conv_w, conv_b, centroids):
    x2 = x.reshape(_B, _C, _HW)
    xf = x.reshape(_B, _HW, _C)
    out = pl.pallas_call(
        _netvlad_kernel,
        grid=(_B,),
        in_specs=[
            pl.BlockSpec((1, _C, _HW), lambda i: (i, 0, 0)),
            pl.BlockSpec((1, _HW, _C), lambda i: (i, 0, 0)),
            pl.BlockSpec((_K, _C), lambda i: (0, 0)),
            pl.BlockSpec((_K, 1), lambda i: (0, 0)),
            pl.BlockSpec((_K, _C), lambda i: (0, 0)),
        ],
        out_specs=pl.BlockSpec((1, _K, _C), lambda i: (i, 0, 0)),
        out_shape=jax.ShapeDtypeStruct((_B, _K, _C), jnp.float32),
        compiler_params=pltpu.CompilerParams(
            dimension_semantics=("parallel",),
            vmem_limit_bytes=56 * 1024 * 1024,
        ),
        name="netvlad_fused",
    )(x2, xf, conv_w, conv_b.reshape(_K, 1), centroids)
    return out.reshape(_B, _K * _C)


# speedup vs baseline: 2.3547x; 2.3547x over previous
import jax
import jax.numpy as jnp
from jax.experimental import pallas as pl
from jax.experimental.pallas import tpu as pltpu

_B, _C, _K, _H, _W = 64, 512, 64, 64, 64
_HW = _H * _W
_R = _HW // _C
_EPS = 1e-12


def _netvlad_kernel(x2_ref, w_ref, b_ref, cent_ref, out_ref):
    x2 = x2_ref[0]
    u = jax.lax.dot_general(
        w_ref[...], x2, (((1,), (0,)), ((), ())),
        preferred_element_type=jnp.float32)
    ssq = jnp.sum(x2 * x2, axis=0, keepdims=True)
    rnorm = 1.0 / jnp.maximum(jnp.sqrt(ssq), _EPS)
    logits = u * rnorm + b_ref[...]
    m = jnp.max(logits, axis=0, keepdims=True)
    e = jnp.exp(logits - m)
    a = e / jnp.sum(e, axis=0, keepdims=True)

    xn_bf = (x2 * rnorm).astype(jnp.bfloat16)
    xfn = xn_bf.reshape(_C, _R, _C).reshape(_HW, _C)

    vlad = jax.lax.dot_general(
        a.astype(jnp.bfloat16), xfn, (((1,), (0,)), ((), ())),
        preferred_element_type=jnp.float32)
    vlad = vlad - jnp.sum(a, axis=1, keepdims=True) * cent_ref[...]
    n1 = jnp.sqrt(jnp.sum(vlad * vlad, axis=1, keepdims=True))
    vlad = vlad / jnp.maximum(n1, _EPS)
    n2 = jnp.sqrt(jnp.sum(vlad * vlad))
    out_ref[0] = vlad / jnp.maximum(n2, _EPS)


def kernel(x, conv_w, conv_b, centroids):
    out = pl.pallas_call(
        _netvlad_kernel,
        grid=(_B,),
        in_specs=[
            pl.BlockSpec((1, _C, _HW), lambda i: (i, 0, 0)),
            pl.BlockSpec((_K, _C), lambda i: (0, 0)),
            pl.BlockSpec((_K, 1), lambda i: (0, 0)),
            pl.BlockSpec((_K, _C), lambda i: (0, 0)),
        ],
        out_specs=pl.BlockSpec((1, _K, _C), lambda i: (i, 0, 0)),
        out_shape=jax.ShapeDtypeStruct((_B, _K, _C), jnp.float32),
        compiler_params=pltpu.CompilerParams(
            dimension_semantics=("parallel",),
            vmem_limit_bytes=56 * 1024 * 1024,
        ),
        name="netvlad_fused",
    )(x.reshape(_B, _C, _HW), conv_w, conv_b.reshape(_K, 1), centroids)
    return out.reshape(_B, _K * _C)
